# baseline (speedup 1.0000x reference)
import functools

import jax
import jax.numpy as jnp
from jax.experimental import pallas as pl
from jax.experimental.pallas import tpu as pltpu


def _body(nmc, x_ref, f_ref, o_ref, acc_ref):
    order = x_ref.shape[0]
    j = pl.program_id(1)
    for a in range(order):
        ya = jnp.dot(f_ref[0, a], x_ref[a], preferred_element_type=jnp.float32)

        @pl.when(j == 0)
        def _init():
            acc_ref[a] = ya

        @pl.when(j != 0)
        def _accum():
            acc_ref[a] = acc_ref[a] + ya

    @pl.when(j == nmc - 1)
    def _finish():
        acc = acc_ref[0]
        for a in range(1, order):
            acc = acc * acc_ref[a]
        o_ref[0, :] = jnp.sum(acc, axis=0)


def kernel(indices_tensor, factors):
    B, order, M = indices_tensor.shape
    R = factors.shape[-1]
    BS = 2048
    MC = 200
    nmc = M // MC

    xt = jnp.transpose(indices_tensor, (1, 2, 0))
    ft = jnp.transpose(factors, (0, 2, 1))
    fc = ft.reshape(order, R, nmc, MC).transpose(2, 0, 1, 3)

    out = pl.pallas_call(
        functools.partial(_body, nmc),
        grid=(B // BS, nmc),
        in_specs=[
            pl.BlockSpec((order, MC, BS), lambda i, j: (0, j, i)),
            pl.BlockSpec((1, order, R, MC), lambda i, j: (j, 0, 0, 0)),
        ],
        out_specs=pl.BlockSpec((1, BS), lambda i, j: (0, i)),
        out_shape=jax.ShapeDtypeStruct((1, B), jnp.float32),
        scratch_shapes=[pltpu.VMEM((order, R, BS), jnp.float32)],
        compiler_params=pltpu.CompilerParams(
            dimension_semantics=("arbitrary", "arbitrary"),
        ),
    )(xt, fc)
    return out[0]


# speedup vs baseline: 1.0350x; 1.0350x over previous
import functools

import jax
import jax.numpy as jnp
from jax.experimental import pallas as pl
from jax.experimental.pallas import tpu as pltpu


def _body(nmc, x_ref, f_ref, o_ref, acc_ref):
    order = x_ref.shape[0]
    j = pl.program_id(0)
    for a in range(order):
        ya = jnp.dot(f_ref[0, a], x_ref[a], preferred_element_type=jnp.float32)

        @pl.when(j == 0)
        def _init():
            acc_ref[a] = ya

        @pl.when(j != 0)
        def _accum():
            acc_ref[a] = acc_ref[a] + ya

    @pl.when(j == nmc - 1)
    def _finish():
        acc = acc_ref[0]
        for a in range(1, order):
            acc = acc * acc_ref[a]
        o_ref[0, :] = jnp.sum(acc, axis=0)


def kernel(indices_tensor, factors):
    B, order, M = indices_tensor.shape
    R = factors.shape[-1]
    MC = 40
    nmc = M // MC

    xt = jnp.transpose(indices_tensor, (1, 2, 0))
    ft = jnp.transpose(factors, (0, 2, 1))
    fc = ft.reshape(order, R, nmc, MC).transpose(2, 0, 1, 3)

    out = pl.pallas_call(
        functools.partial(_body, nmc),
        grid=(nmc,),
        in_specs=[
            pl.BlockSpec((order, MC, B), lambda j: (0, j, 0)),
            pl.BlockSpec((1, order, R, MC), lambda j: (j, 0, 0, 0)),
        ],
        out_specs=pl.BlockSpec((1, B), lambda j: (0, 0)),
        out_shape=jax.ShapeDtypeStruct((1, B), jnp.float32),
        scratch_shapes=[pltpu.VMEM((order, R, B), jnp.float32)],
        compiler_params=pltpu.CompilerParams(
            dimension_semantics=("arbitrary",),
        ),
    )(xt, fc)
    return out[0]


# speedup vs baseline: 1.0788x; 1.0423x over previous
import jax
import jax.numpy as jnp
from jax.experimental import pallas as pl
from jax.experimental.pallas import tpu as pltpu


def _body(x_ref, f_ref, o_ref):
    order = x_ref.shape[0]
    acc = None
    for a in range(order):
        ya = jnp.dot(f_ref[a], x_ref[a], preferred_element_type=jnp.float32)
        acc = ya if acc is None else acc * ya
    o_ref[0, :] = jnp.sum(acc, axis=0)


def kernel(indices_tensor, factors):
    B, order, M = indices_tensor.shape
    R = factors.shape[-1]
    BS = 512

    xt = jnp.transpose(indices_tensor, (1, 2, 0))
    ft = jnp.transpose(factors, (0, 2, 1))

    out = pl.pallas_call(
        _body,
        grid=(B // BS,),
        in_specs=[
            pl.BlockSpec((order, M, BS), lambda i: (0, 0, i)),
            pl.BlockSpec((order, R, M), lambda i: (0, 0, 0)),
        ],
        out_specs=pl.BlockSpec((1, BS), lambda i: (0, i)),
        out_shape=jax.ShapeDtypeStruct((1, B), jnp.float32),
        compiler_params=pltpu.CompilerParams(
            dimension_semantics=("arbitrary",),
        ),
    )(xt, ft)
    return out[0]
